# Initial kernel scaffold; baseline (speedup 1.0000x reference)
import jax
import jax.numpy as jnp
from jax.experimental import pallas as pl


def kernel(pos, p, cell_type, edge_index):
    raise NotImplementedError("write your pallas kernel here")


# speedup vs baseline: 75.8407x; 75.8407x over previous
import jax
import jax.numpy as jnp
from jax import lax
from jax.experimental import pallas as pl
from jax.experimental.pallas import tpu as pltpu
from jax.experimental.pallas import tpu_sc as plsc

N_NODES = 100000
N_EDGES = 6400000
LANES = 16
NC = 2
NS = 16
NW = NC * NS
ROW_W = 128
CHUNK_ROWS = 16
CHUNK = CHUNK_ROWS * ROW_W
EIDX_ROWS = N_EDGES // ROW_W
N_CHUNKS = N_EDGES // CHUNK
BASE_CHUNKS = N_CHUNKS // NW
EXTRA = N_CHUNKS - BASE_CHUNKS * NW
GROUPS = CHUNK // LANES
ACC_TILE_ROWS = 6256
ACC_LAST_ROWS = N_NODES - 15 * ACC_TILE_ROWS

_MAGIC = 0x5F3759DF

_DNUMS = lax.GatherDimensionNumbers(
    offset_dims=(), collapsed_slice_dims=(0,), start_index_map=(0,))


def _vgather(table_vec, idx):
  return lax.gather(table_vec, idx[:, None], _DNUMS, (1,),
                    mode=lax.GatherScatterMode.PROMISE_IN_BOUNDS)


def _force_kernel(xs_h, ys_h, zs_h, ct_h, praw, zrows, dst_f, src_f,
                  out,
                  di1, si1, di2, gxi, gyi, gzi, gct, gxj, gyj, gzj,
                  stx, sty, stz, praw_v,
                  xs, ys, zs, cts, accx, accy, accz,
                  sem_g, sem_s):
  cid_c = lax.axis_index("c")
  sid = lax.axis_index("s")
  wid = sid * NC + cid_c

  r0_off = sid * ACC_TILE_ROWS

  def _stage(total):
    done = 0
    while done < total:
      n = min(2048, total - done)
      sl = pl.ds(r0_off + done, n)
      pltpu.sync_copy(xs_h.at[sl], xs.at[sl])
      pltpu.sync_copy(ys_h.at[sl], ys.at[sl])
      pltpu.sync_copy(zs_h.at[sl], zs.at[sl])
      pltpu.sync_copy(ct_h.at[sl], cts.at[sl])
      zsl = pl.ds(0, n)
      pltpu.sync_copy(zrows.at[zsl], accx.at[sl])
      pltpu.sync_copy(zrows.at[zsl], accy.at[sl])
      pltpu.sync_copy(zrows.at[zsl], accz.at[sl])
      done += n

  @pl.when(sid < 15)
  def _():
    _stage(ACC_TILE_ROWS)

  @pl.when(sid == 15)
  def _():
    _stage(ACC_LAST_ROWS)

  pltpu.sync_copy(praw, praw_v)
  k_rep = praw_v[0]
  r0_t = praw_v[1]
  kadh = praw_v[2]
  r_on = praw_v[3]
  delta = praw_v[4]
  mu_f = praw_v[5]
  A_v = mu_f * k_rep
  B_v = mu_f * kadh
  invd_v = 1.0 / jnp.maximum(delta, 1e-8)

  plsc.subcore_barrier()

  nw = jnp.where(wid < EXTRA, BASE_CHUNKS + 1, BASE_CHUNKS)

  def _chunk(g, _):
    cid = wid + NW * g
    ebase = cid * CHUNK
    pltpu.sync_copy(dst_f.at[pl.ds(ebase, CHUNK)], di1)
    pltpu.sync_copy(src_f.at[pl.ds(ebase, CHUNK)], si1)

    cps = [
        pltpu.async_copy(xs.at[di1], gxi, sem_g),
        pltpu.async_copy(ys.at[di1], gyi, sem_g),
        pltpu.async_copy(zs.at[di1], gzi, sem_g),
        pltpu.async_copy(cts.at[di1], gct, sem_g),
        pltpu.async_copy(xs.at[si1], gxj, sem_g),
        pltpu.async_copy(ys.at[si1], gyj, sem_g),
        pltpu.async_copy(zs.at[si1], gzj, sem_g),
    ]
    for cp in cps:
      cp.wait()

    def _group(j, _):
      l = j * LANES
      sl = pl.ds(l, LANES)
      m = j // (ROW_W // LANES)
      lofs = (j % (ROW_W // LANES)) * LANES
      xi = gxi[sl]
      yi = gyi[sl]
      zi = gzi[sl]
      ct = gct[sl]
      xj = gxj[sl]
      yj = gyj[sl]
      zj = gzj[sl]
      dv = di1[sl]
      sv = si1[sl]
      di2[m, pl.ds(lofs, LANES)] = dv

      dx = xj - xi
      dy = yj - yi
      dz = zj - zi
      r2 = dx * dx + dy * dy + dz * dz
      r2s = jnp.maximum(r2, 1e-30)
      yv = plsc.bitcast(_MAGIC - (plsc.bitcast(r2s, jnp.int32) >> 1),
                        jnp.float32)
      h = 0.5 * r2s
      yv = yv * (1.5 - h * yv * yv)
      yv = yv * (1.5 - h * yv * yv)
      yv = yv * (1.5 - h * yv * yv)
      r = r2s * yv
      inv_rs = jnp.minimum(yv, 1e8)

      A = _vgather(A_v, ct)
      B = _vgather(B_v, ct)
      r0v = _vgather(r0_t, ct)
      ronv = _vgather(r_on, ct)
      invd = _vgather(invd_v, ct)

      rel = r - r0v
      frep = A * jnp.maximum(-rel, 0.0)
      e1 = jnp.exp(-(rel * invd))
      e2 = jnp.exp((r - ronv) * invd)
      den = (1.0 + e1) * (1.0 + e2)
      coef = (B * rel / den - frep) * inv_rs
      coef = jnp.where(sv == dv, 0.0, coef)

      stx[sl] = coef * dx
      sty[sl] = coef * dy
      stz[sl] = coef * dz
      return 0

    lax.fori_loop(0, GROUPS, _group, 0)

    def _scat(m, _):
      idx = di2.at[m]
      rsl = pl.ds(m * ROW_W, ROW_W)
      pltpu.async_copy(stx.at[rsl], accx.at[idx], sem_s, add=True)
      pltpu.async_copy(sty.at[rsl], accy.at[idx], sem_s, add=True)
      pltpu.async_copy(stz.at[rsl], accz.at[idx], sem_s, add=True)
      return 0
    lax.fori_loop(0, CHUNK_ROWS, _scat, 0)

    def _sdrain(m, _):
      idx = di2.at[m]
      rsl = pl.ds(m * ROW_W, ROW_W)
      pltpu.make_async_copy(stx.at[rsl], accx.at[idx], sem_s).wait()
      pltpu.make_async_copy(sty.at[rsl], accy.at[idx], sem_s).wait()
      pltpu.make_async_copy(stz.at[rsl], accz.at[idx], sem_s).wait()
      return 0
    lax.fori_loop(0, CHUNK_ROWS, _sdrain, 0)
    return 0

  lax.fori_loop(0, nw, _chunk, 0)
  plsc.subcore_barrier()

  def _dump(total):
    done = 0
    while done < total:
      n = min(2048, total - done)
      sl = pl.ds(r0_off + done, n)
      pltpu.sync_copy(accx.at[sl], out.at[pl.ds(cid_c * 3 * N_NODES + r0_off + done, n)])
      pltpu.sync_copy(accy.at[sl], out.at[pl.ds((cid_c * 3 + 1) * N_NODES + r0_off + done, n)])
      pltpu.sync_copy(accz.at[sl], out.at[pl.ds((cid_c * 3 + 2) * N_NODES + r0_off + done, n)])
      done += n

  @pl.when(sid < 15)
  def _():
    _dump(ACC_TILE_ROWS)

  @pl.when(sid == 15)
  def _():
    _dump(ACC_LAST_ROWS)


@jax.jit
def _run(xs, ys, zs, cts, praw, zrows, dst_f, src_f):
  mesh = plsc.VectorSubcoreMesh(core_axis_name="c", subcore_axis_name="s",
                                num_cores=NC, num_subcores=NS)
  f = pl.kernel(
      _force_kernel,
      out_type=jax.ShapeDtypeStruct((2 * 3 * N_NODES,), jnp.float32),
      mesh=mesh,
      compiler_params=pltpu.CompilerParams(needs_layout_passes=False,
                                           use_tc_tiling_on_sc=False),
      scratch_types=[
          pltpu.VMEM((CHUNK,), jnp.int32),
          pltpu.VMEM((CHUNK,), jnp.int32),
          pltpu.VMEM((CHUNK_ROWS, ROW_W), jnp.int32),
          pltpu.VMEM((CHUNK,), jnp.float32),
          pltpu.VMEM((CHUNK,), jnp.float32),
          pltpu.VMEM((CHUNK,), jnp.float32),
          pltpu.VMEM((CHUNK,), jnp.int32),
          pltpu.VMEM((CHUNK,), jnp.float32),
          pltpu.VMEM((CHUNK,), jnp.float32),
          pltpu.VMEM((CHUNK,), jnp.float32),
          pltpu.VMEM((CHUNK,), jnp.float32),
          pltpu.VMEM((CHUNK,), jnp.float32),
          pltpu.VMEM((CHUNK,), jnp.float32),
          pltpu.VMEM((8, LANES), jnp.float32),
          pltpu.VMEM_SHARED((N_NODES,), jnp.float32),
          pltpu.VMEM_SHARED((N_NODES,), jnp.float32),
          pltpu.VMEM_SHARED((N_NODES,), jnp.float32),
          pltpu.VMEM_SHARED((N_NODES,), jnp.int32),
          pltpu.VMEM_SHARED((N_NODES,), jnp.float32),
          pltpu.VMEM_SHARED((N_NODES,), jnp.float32),
          pltpu.VMEM_SHARED((N_NODES,), jnp.float32),
          pltpu.SemaphoreType.DMA,
          pltpu.SemaphoreType.DMA,
      ],
  )
  return f(xs, ys, zs, cts, praw, zrows, dst_f, src_f)


def kernel(pos, p, cell_type, edge_index):
  xs = pos[:, 0]
  ys = pos[:, 1]
  zs = pos[:, 2]
  cts = cell_type.astype(jnp.int32)
  praw = jnp.zeros((8, LANES), jnp.float32).at[:6, :5].set(p.T)
  zrows = jnp.zeros((2048,), jnp.float32)
  dst_f = edge_index[0]
  src_f = edge_index[1]
  out = _run(xs, ys, zs, cts, praw, zrows, dst_f, src_f)
  o = out.reshape(2, 3, N_NODES)
  return (o[0] + o[1]).T
